# baseline (speedup 1.0000x reference)
import functools

import jax
import jax.numpy as jnp
from jax import lax
from jax.experimental import pallas as pl
from jax.experimental.pallas import tpu as pltpu
from jax.experimental.pallas import tpu_sc as plsc

D = 260
NGROUP = 26
GSIZE = 10
LANES = 16

NUM_CORES = 2
NUM_SUBCORES = 16
NW = NUM_CORES * NUM_SUBCORES

ROWS_BLK = 40
NRG = 3
PITCH = 280
LIN_W = ROWS_BLK * PITCH
GLEN = LIN_W - 256


def _tail_cols(iota):
    return jnp.full((LANES,), 256, jnp.int32) + jnp.minimum(iota, 3)


def _detile_row(in_2d, lin, r, iota):
    base = r * PITCH
    for k in range(16):
        lin[pl.ds(base + k * LANES, LANES)] = in_2d[r, pl.ds(k * LANES, LANES)]
    tail = plsc.load_gather(in_2d, [jnp.full((LANES,), r, jnp.int32), _tail_cols(iota)])
    lin[pl.ds(base + 256, LANES)] = tail


def _retile_row(lin, out_2d, r, iota):
    base = r * PITCH
    for k in range(16):
        out_2d[r, pl.ds(k * LANES, LANES)] = lin[pl.ds(base + k * LANES, LANES)]
    tail = plsc.load_gather(lin, [jnp.full((LANES,), base, jnp.int32) + _tail_cols(iota)])
    plsc.store_scatter(out_2d, [jnp.full((LANES,), r, jnp.int32), _tail_cols(iota)], tail)


def _argmax_tree(vals):
    def duel(av, ai, bv, bi):
        gt = bv > av
        return jnp.where(gt, bv, av), jnp.where(gt, bi, ai)

    idx = [jnp.full((LANES,), j, jnp.int32) for j in range(GSIZE)]
    w = [duel(vals[2 * p], idx[2 * p], vals[2 * p + 1], idx[2 * p + 1])
         for p in range(5)]
    a = duel(w[0][0], w[0][1], w[1][0], w[1][1])
    b = duel(w[2][0], w[2][1], w[3][0], w[3][1])
    d = duel(a[0], a[1], b[0], b[1])
    f = duel(d[0], d[1], w[4][0], w[4][1])
    return f[1]


def _process_rowgroup(lin_in, lin_out, rowoff):
    one_f = jnp.full((LANES,), 1.0, jnp.float32)
    zero_f = jnp.zeros((LANES,), jnp.float32)
    rowoffs = [rowoff + m for m in range(8)]
    for g in range(NGROUP):
        c0 = g * GSIZE
        vals = [
            plsc.load_gather(
                lin_in.at[pl.ds(((c0 + j) // 8) * 8, GLEN)],
                [rowoffs[(c0 + j) % 8]],
            )
            for j in range(GSIZE)
        ]
        bi = _argmax_tree(vals)
        for j in range(GSIZE):
            oh = jnp.where(bi == jnp.full((LANES,), j, jnp.int32), one_f, zero_f)
            plsc.store_scatter(
                lin_out.at[pl.ds(((c0 + j) // 8) * 8, GLEN)],
                [rowoffs[(c0 + j) % 8]],
                oh,
            )


def _make_kernel(n_rows):
    rows_per_w = n_rows // NW
    nblk = rows_per_w // ROWS_BLK
    mesh = plsc.VectorSubcoreMesh(core_axis_name="c", subcore_axis_name="s")

    @functools.partial(
        pl.kernel,
        mesh=mesh,
        out_type=jax.ShapeDtypeStruct((n_rows, D), jnp.float32),
        compiler_params=pltpu.CompilerParams(
            use_tc_tiling_on_sc=True, needs_layout_passes=False
        ),
        scratch_types=[
            pltpu.VMEM((2, ROWS_BLK, D), jnp.float32),
            pltpu.VMEM((2, ROWS_BLK, D), jnp.float32),
            pltpu.VMEM((LIN_W,), jnp.float32),
            pltpu.VMEM((LIN_W,), jnp.float32),
            pltpu.SemaphoreType.DMA,
            pltpu.SemaphoreType.DMA,
        ],
    )
    def onehot_argmax(x_hbm, out_hbm, in_v, out_v, lin_in, lin_out, in_sem, out_sem):
        wid = lax.axis_index("s") * NUM_CORES + lax.axis_index("c")
        row0 = wid * rows_per_w

        def in_copy(i, slot):
            src = x_hbm.at[pl.ds(row0 + i * ROWS_BLK, ROWS_BLK), :]
            return pltpu.make_async_copy(src, in_v.at[slot], in_sem)

        def out_copy(i, slot):
            dst = out_hbm.at[pl.ds(row0 + i * ROWS_BLK, ROWS_BLK), :]
            return pltpu.make_async_copy(out_v.at[slot], dst, out_sem)

        iota = lax.iota(jnp.int32, LANES)
        in_copy(0, 0).start()

        def blk(i, _):
            slot = lax.rem(i, 2)
            nxt = 1 - slot

            @pl.when(i + 1 < nblk)
            def _():
                in_copy(i + 1, nxt).start()

            in_copy(i, slot).wait()

            @pl.when(i >= 2)
            def _():
                out_copy(i - 2, slot).wait()

            in_2d = in_v.at[slot]
            out_2d = out_v.at[slot]

            for r in range(ROWS_BLK):
                _detile_row(in_2d, lin_in, r, iota)

            def rowgrp(rg, _):
                start = jnp.minimum(rg * LANES, ROWS_BLK - LANES)
                rowoff = (start + iota) * PITCH
                _process_rowgroup(lin_in, lin_out, rowoff)
                return 0

            lax.fori_loop(0, NRG, rowgrp, 0)

            for r in range(ROWS_BLK):
                _retile_row(lin_out, out_2d, r, iota)

            out_copy(i, slot).start()
            return 0

        lax.fori_loop(0, nblk, blk, 0)
        out_copy(nblk - 2, lax.rem(nblk - 2, 2)).wait()
        out_copy(nblk - 1, lax.rem(nblk - 1, 2)).wait()

    return onehot_argmax


def kernel(x):
    nb, nt, d = x.shape
    out = _make_kernel(nb * nt)(x.reshape(nb * nt, d))
    return out.reshape(nb, nt, d)


# speedup vs baseline: 1.2011x; 1.2011x over previous
import functools

import jax
import jax.numpy as jnp
from jax import lax
from jax.experimental import pallas as pl
from jax.experimental.pallas import tpu as pltpu
from jax.experimental.pallas import tpu_sc as plsc

D = 260
NGROUP = 26
GSIZE = 10
LANES = 16

NUM_CORES = 2
NUM_SUBCORES = 16
NW = NUM_CORES * NUM_SUBCORES

ROWS_BLK = 64


def _argmax_tree(vals):
    def duel(av, ai, bv, bi):
        gt = bv > av
        return jnp.where(gt, bv, av), jnp.where(gt, bi, ai)

    idx = [jnp.full((LANES,), j, jnp.int32) for j in range(GSIZE)]
    w = [duel(vals[2 * p], idx[2 * p], vals[2 * p + 1], idx[2 * p + 1])
         for p in range(5)]
    a = duel(w[0][0], w[0][1], w[1][0], w[1][1])
    b = duel(w[2][0], w[2][1], w[3][0], w[3][1])
    d = duel(a[0], a[1], b[0], b[1])
    f = duel(d[0], d[1], w[4][0], w[4][1])
    return f[1]


def _process_rowgroup(in_v, out_v, rows):
    one_f = jnp.full((LANES,), 1.0, jnp.float32)
    zero_f = jnp.zeros((LANES,), jnp.float32)
    for g in range(NGROUP):
        c0 = g * GSIZE
        vals = []
        for j in range(GSIZE):
            cidx = jnp.full((LANES,), c0 + j, jnp.int32)
            vals.append(plsc.load_gather(in_v, [rows, cidx]))
        bi = _argmax_tree(vals)
        for j in range(GSIZE):
            oh = jnp.where(bi == jnp.full((LANES,), j, jnp.int32), one_f, zero_f)
            cidx = jnp.full((LANES,), c0 + j, jnp.int32)
            plsc.store_scatter(out_v, [rows, cidx], oh)


def _make_kernel(n_rows):
    rows_per_w = n_rows // NW
    nblk = rows_per_w // ROWS_BLK
    mesh = plsc.VectorSubcoreMesh(core_axis_name="c", subcore_axis_name="s")

    @functools.partial(
        pl.kernel,
        mesh=mesh,
        out_type=jax.ShapeDtypeStruct((n_rows, D), jnp.float32),
        compiler_params=pltpu.CompilerParams(
            use_tc_tiling_on_sc=False, needs_layout_passes=False
        ),
        scratch_types=[
            pltpu.VMEM((2, ROWS_BLK, D), jnp.float32),
            pltpu.VMEM((2, ROWS_BLK, D), jnp.float32),
            pltpu.SemaphoreType.DMA,
            pltpu.SemaphoreType.DMA,
        ],
    )
    def onehot_argmax(x_hbm, out_hbm, in_v, out_v, in_sem, out_sem):
        wid = lax.axis_index("s") * NUM_CORES + lax.axis_index("c")
        row0 = wid * rows_per_w

        def in_copy(i, slot):
            src = x_hbm.at[pl.ds(row0 + i * ROWS_BLK, ROWS_BLK), :]
            return pltpu.make_async_copy(src, in_v.at[slot], in_sem)

        def out_copy(i, slot):
            dst = out_hbm.at[pl.ds(row0 + i * ROWS_BLK, ROWS_BLK), :]
            return pltpu.make_async_copy(out_v.at[slot], dst, out_sem)

        iota = lax.iota(jnp.int32, LANES)

        in_copy(0, 0).start()

        def blk(i, _):
            slot = lax.rem(i, 2)
            nxt = 1 - slot

            @pl.when(i + 1 < nblk)
            def _():
                in_copy(i + 1, nxt).start()

            in_copy(i, slot).wait()

            @pl.when(i >= 2)
            def _():
                out_copy(i - 2, slot).wait()

            def rowgrp(rg, _):
                rows = rg * LANES + iota
                _process_rowgroup(in_v.at[slot], out_v.at[slot], rows)
                return 0

            lax.fori_loop(0, ROWS_BLK // LANES, rowgrp, 0)

            out_copy(i, slot).start()
            return 0

        lax.fori_loop(0, nblk, blk, 0)

        out_copy(nblk - 2, lax.rem(nblk - 2, 2)).wait()
        out_copy(nblk - 1, lax.rem(nblk - 1, 2)).wait()

    return onehot_argmax


def kernel(x):
    b, t, d = x.shape
    xf = x.reshape(b * t, d)
    out = _make_kernel(b * t)(xf)
    return out.reshape(b, t, d)
